# baseline (speedup 1.0000x reference)
import functools

import jax
import jax.numpy as jnp
from jax import lax
from jax.experimental import pallas as pl
from jax.experimental.pallas import tpu as pltpu
from jax.experimental.pallas import tpu_sc as plsc

INPUT_SIZE = 2048
NUM_EXPERTS = 64
TOP_K = 8
TOKEN_TILE = 1024
NEG_INF = float("-inf")


def _matmul_kernel(x_ref, w_ref, b_ref, out_ref):
    logits = lax.dot_general(
        x_ref[...],
        w_ref[...],
        dimension_numbers=(((1,), (1,)), ((), ())),
        preferred_element_type=jnp.float32,
    )
    out_ref[...] = logits + b_ref[...]


def _tc_logits(x2, W, b2, tokens, D):
    n_tiles = tokens // TOKEN_TILE
    return pl.pallas_call(
        _matmul_kernel,
        grid=(n_tiles,),
        in_specs=[
            pl.BlockSpec((TOKEN_TILE, D), lambda i: (i, 0)),
            pl.BlockSpec((NUM_EXPERTS, D), lambda i: (0, 0)),
            pl.BlockSpec((1, NUM_EXPERTS), lambda i: (0, 0)),
        ],
        out_specs=pl.BlockSpec((TOKEN_TILE, NUM_EXPERTS), lambda i: (i, 0)),
        out_shape=jax.ShapeDtypeStruct((tokens, NUM_EXPERTS), jnp.float32),
    )(x2, W, b2)


def _sc_route(logits):
    tokens = logits.shape[0] // NUM_EXPERTS
    info = plsc.get_sparse_core_info()
    nc, ns, L = info.num_cores, info.num_subcores, info.num_lanes
    nw = nc * ns
    tpw = tokens // nw
    n_groups = tpw // L
    mesh = plsc.VectorSubcoreMesh(core_axis_name="c", subcore_axis_name="s")

    @functools.partial(
        pl.kernel,
        mesh=mesh,
        out_type=jax.ShapeDtypeStruct((tokens * NUM_EXPERTS,), jnp.float32),
        scratch_types=[
            pltpu.VMEM((tpw * NUM_EXPERTS,), jnp.float32),
            pltpu.VMEM((tpw * NUM_EXPERTS,), jnp.float32),
        ],
        compiler_params=pltpu.CompilerParams(needs_layout_passes=False),
    )
    def _route(lg_hbm, out_hbm, lg_v, out_v):
        wid = lax.axis_index("s") * nc + lax.axis_index("c")
        base = wid * tpw * NUM_EXPERTS
        pltpu.sync_copy(lg_hbm.at[pl.ds(base, tpw * NUM_EXPERTS)], lg_v)

        lane = lax.broadcasted_iota(jnp.int32, (L,), 0)

        def group_body(g, _):
            rowbase = (lane + g * L) * NUM_EXPERTS
            idxs = [rowbase + e for e in range(NUM_EXPERTS)]
            work = [plsc.load_gather(lg_v, [idxs[e]]) for e in range(NUM_EXPERTS)]

            def tree_max(vals):
                while len(vals) > 1:
                    vals = [
                        jnp.maximum(vals[2 * i], vals[2 * i + 1])
                        for i in range(len(vals) // 2)
                    ]
                return vals[0]

            m0 = None
            m = None
            for _r in range(TOP_K):
                m = tree_max(work)
                if m0 is None:
                    m0 = m
                if _r < TOP_K - 1:
                    work = [jnp.where(w == m, NEG_INF, w) for w in work]

            den = jnp.zeros((L,), jnp.float32)
            ps = []
            for e in range(NUM_EXPERTS):
                v = plsc.load_gather(lg_v, [idxs[e]])
                p = jnp.where(v >= m, jnp.exp(v - m0), jnp.float32(0.0))
                den = den + p
                ps.append(p)
            rden = 1.0 / den
            for e in range(NUM_EXPERTS):
                plsc.store_scatter(out_v, [idxs[e]], ps[e] * rden)
            return _

        lax.fori_loop(0, n_groups, group_body, 0)
        pltpu.sync_copy(out_v, out_hbm.at[pl.ds(base, tpw * NUM_EXPERTS)])

    return _route(logits)


@jax.jit
def kernel(x, W, b):
    B, S, D = x.shape
    tokens = B * S
    x2 = x.reshape(tokens, D)
    b2 = b.reshape(1, NUM_EXPERTS)
    logits = _tc_logits(x2, W, b2, tokens, D)
    routed = _sc_route(logits.reshape(tokens * NUM_EXPERTS))
    return routed.reshape(B, S, NUM_EXPERTS)


# speedup vs baseline: 2.3596x; 2.3596x over previous
import jax
import jax.numpy as jnp
from jax.experimental import pallas as pl
from jax.experimental.pallas import tpu as pltpu

INPUT_SIZE = 2048
NUM_EXPERTS = 64
TOP_K = 8
TOKEN_TILE = 1024


def _topk_softmax(logits):
    neg_inf = jnp.float32(-jnp.inf)
    work = logits
    m0 = None
    for _ in range(TOP_K):
        m = jnp.max(work, axis=-1, keepdims=True)
        if m0 is None:
            m0 = m
        work = jnp.where(work == m, neg_inf, work)
    q = jnp.where(logits >= m, logits, neg_inf)
    p = jnp.exp(q - m0)
    den = jnp.sum(p, axis=-1, keepdims=True)
    return p * (1.0 / den)


def _gate_kernel(n_tiles, xa_ref, xb_ref, wt_ref, b_ref, out_ref, lg_ref):
    i = pl.program_id(0)
    slot = jax.lax.rem(i, 2)
    half = INPUT_SIZE // 2
    dn = (((1,), (1,)), ((), ()))
    logits = jax.lax.dot_general(
        xa_ref[...], wt_ref[:, :half], dn, preferred_element_type=jnp.float32
    ) + jax.lax.dot_general(
        xb_ref[...], wt_ref[:, half:], dn, preferred_element_type=jnp.float32
    )
    routed = _topk_softmax(lg_ref[1 - slot])
    lg_ref[slot] = logits + b_ref[...]
    out_ref[...] = routed


@jax.jit
def kernel(x, W, b):
    B, S, D = x.shape
    tokens = B * S
    x2 = x.reshape(tokens, D)
    b2 = b.reshape(1, NUM_EXPERTS)

    n_tiles = tokens // TOKEN_TILE
    import functools

    out = pl.pallas_call(
        functools.partial(_gate_kernel, n_tiles),
        grid=(n_tiles + 1,),
        in_specs=[
            pl.BlockSpec(
                (TOKEN_TILE, D // 2), lambda i: (jnp.minimum(i, n_tiles - 1), 0)
            ),
            pl.BlockSpec(
                (TOKEN_TILE, D // 2), lambda i: (jnp.minimum(i, n_tiles - 1), 1)
            ),
            pl.BlockSpec((NUM_EXPERTS, D), lambda i: (0, 0)),
            pl.BlockSpec((1, NUM_EXPERTS), lambda i: (0, 0)),
        ],
        out_specs=pl.BlockSpec(
            (TOKEN_TILE, NUM_EXPERTS), lambda i: (jnp.maximum(i - 1, 0), 0)
        ),
        out_shape=jax.ShapeDtypeStruct((tokens, NUM_EXPERTS), jnp.float32),
        scratch_shapes=[pltpu.VMEM((2, TOKEN_TILE, NUM_EXPERTS), jnp.float32)],
    )(x2, x2, W, b2)
    return out.reshape(B, S, NUM_EXPERTS)


# speedup vs baseline: 2.3798x; 1.0085x over previous
import jax
import jax.numpy as jnp
from jax.experimental import pallas as pl
from jax.experimental.pallas import tpu as pltpu

INPUT_SIZE = 2048
NUM_EXPERTS = 64
TOP_K = 8
TOKEN_TILE = 1024


def _topk_softmax(logits):
    neg_inf = jnp.float32(-jnp.inf)
    work = logits
    m0 = None
    for _ in range(TOP_K):
        m = jnp.max(work, axis=-1, keepdims=True)
        if m0 is None:
            m0 = m
        work = jnp.where(work == m, neg_inf, work)
    q = jnp.where(logits >= m, logits, neg_inf)
    p = jnp.exp(q - m0)
    den = jnp.sum(p, axis=-1, keepdims=True)
    return p * (1.0 / den)


def _gate_kernel(n_tiles, x_ref, wt_ref, b_ref, out_ref, lg_ref):
    i = pl.program_id(0)
    slot = jax.lax.rem(i, 2)
    logits = jax.lax.dot_general(
        x_ref[...],
        wt_ref[...],
        dimension_numbers=(((1,), (1,)), ((), ())),
        preferred_element_type=jnp.float32,
    )
    routed = _topk_softmax(lg_ref[1 - slot])
    lg_ref[slot] = logits + b_ref[...]
    out_ref[...] = routed


@jax.jit
def kernel(x, W, b):
    B, S, D = x.shape
    tokens = B * S
    x2 = x.reshape(tokens, D)
    b2 = b.reshape(1, NUM_EXPERTS)

    n_tiles = tokens // TOKEN_TILE
    import functools

    out = pl.pallas_call(
        functools.partial(_gate_kernel, n_tiles),
        grid=(n_tiles + 1,),
        in_specs=[
            pl.BlockSpec((TOKEN_TILE, D), lambda i: (jnp.minimum(i, n_tiles - 1), 0)),
            pl.BlockSpec((NUM_EXPERTS, D), lambda i: (0, 0)),
            pl.BlockSpec((1, NUM_EXPERTS), lambda i: (0, 0)),
        ],
        out_specs=pl.BlockSpec(
            (TOKEN_TILE, NUM_EXPERTS), lambda i: (jnp.maximum(i - 1, 0), 0)
        ),
        out_shape=jax.ShapeDtypeStruct((tokens, NUM_EXPERTS), jnp.float32),
        scratch_shapes=[pltpu.VMEM((2, TOKEN_TILE, NUM_EXPERTS), jnp.float32)],
    )(x2, W, b2)
    return out.reshape(B, S, NUM_EXPERTS)
